# baseline (speedup 1.0000x reference)
import functools

import jax
import jax.numpy as jnp
from jax import lax
from jax.experimental import pallas as pl
from jax.experimental.pallas import tpu as pltpu
from jax.experimental.pallas import tpu_sc as plsc

N = 10000
E = 320000
D = 128

NC = 2
NS = 16
NW = NC * NS
E_PER_TILE = E // NW
CHUNK = 80
NCHUNK = E_PER_TILE // CHUNK
NPAD = 10240
ROWS_PER_TILE = NPAD // NS
WB = 32


GROUPS = ((0, 64), (64, NCHUNK - 64))
GBUF = 64 * CHUNK


def _sc_agg_body(x_hbm, ei_hbm, out_hbm,
                 acc_sh, src_v, dst_v, rows_v, wb_v, sem_g, sem_s):
    c = lax.axis_index("c")
    s = lax.axis_index("s")
    w = s * NC + c

    def _zrow(i, _):
        for j in range(D // 16):
            wb_v[i, pl.ds(j * 16, 16)] = jnp.zeros((16,), jnp.float32)
        return 0
    lax.fori_loop(0, WB, _zrow, 0)
    row0 = s * ROWS_PER_TILE
    nz = ROWS_PER_TILE // WB
    for k in range(nz):
        pltpu.async_copy(wb_v, acc_sh.at[pl.ds(row0 + k * WB, WB)], sem_s)
    e00 = w * E_PER_TILE
    pltpu.sync_copy(ei_hbm.at[pl.ds(e00, GBUF)], src_v)
    pltpu.sync_copy(ei_hbm.at[pl.ds(E + e00, GBUF)], dst_v)
    pltpu.async_copy(x_hbm.at[src_v.at[pl.ds(0, CHUNK)]], rows_v.at[0], sem_g)
    for k in range(nz):
        pltpu.make_async_copy(wb_v, acc_sh.at[pl.ds(row0, WB)], sem_s).wait()
    plsc.subcore_barrier()

    def _gather_wait(b):
        pltpu.make_async_copy(x_hbm.at[pl.ds(0, CHUNK)], rows_v.at[b], sem_g).wait()

    def _scatter_wait(b):
        pltpu.make_async_copy(rows_v.at[b], acc_sh.at[pl.ds(0, CHUNK)], sem_s).wait()

    for base, cnt in GROUPS:
        e0 = w * E_PER_TILE + base * CHUNK
        if base:
            pltpu.sync_copy(ei_hbm.at[pl.ds(e0, cnt * CHUNK)],
                            src_v.at[pl.ds(0, cnt * CHUNK)])
            pltpu.sync_copy(ei_hbm.at[pl.ds(E + e0, cnt * CHUNK)],
                            dst_v.at[pl.ds(0, cnt * CHUNK)])
            pltpu.async_copy(x_hbm.at[src_v.at[pl.ds(0, CHUNK)]], rows_v.at[0],
                             sem_g)

        def _chunk(i, _):
            b = lax.rem(i, 2)

            @pl.when(i > 0)
            def _():
                _scatter_wait(1 - b)

            @pl.when(i < cnt - 1)
            def _():
                pltpu.async_copy(
                    x_hbm.at[src_v.at[pl.ds((i + 1) * CHUNK, CHUNK)]],
                    rows_v.at[1 - b], sem_g)
            _gather_wait(b)
            pltpu.async_copy(rows_v.at[b],
                             acc_sh.at[dst_v.at[pl.ds(i * CHUNK, CHUNK)]],
                             sem_s, add=True)
            return 0
        lax.fori_loop(0, cnt, _chunk, 0)
        _scatter_wait(lax.rem(cnt - 1, 2))
    plsc.subcore_barrier()

    WBH = 128
    nw = ROWS_PER_TILE // WBH
    for k in range(nw):
        r = row0 + k * WBH
        pltpu.async_copy(acc_sh.at[pl.ds(r, WBH)], out_hbm.at[c, pl.ds(r, WBH)],
                         sem_g)
    for k in range(nw):
        pltpu.make_async_copy(acc_sh.at[pl.ds(row0, WBH)],
                              out_hbm.at[c, pl.ds(row0, WBH)], sem_g).wait()


@jax.jit
def _sc_aggregate(x, ei4):
    mesh = plsc.VectorSubcoreMesh(core_axis_name="c", subcore_axis_name="s")
    return pl.kernel(
        _sc_agg_body,
        out_type=jax.ShapeDtypeStruct((NC, NPAD, D), jnp.float32),
        mesh=mesh,
        scratch_types=[
            pltpu.VMEM_SHARED((NPAD, D), jnp.float32),
            pltpu.VMEM((GBUF,), jnp.int32),
            pltpu.VMEM((GBUF,), jnp.int32),
            pltpu.VMEM((2, CHUNK, D), jnp.float32),
            pltpu.VMEM((WB, D), jnp.float32),
            pltpu.SemaphoreType.DMA,
            pltpu.SemaphoreType.DMA,
        ],
    )(x, ei4)


def _tc_body(x_ref, p0_ref, p1_ref, w1_ref, b1_ref, wff1_ref, wff2_ref, o_ref):
    agg = p0_ref[0] + p1_ref[0]
    pre = jnp.dot(agg, w1_ref[...], preferred_element_type=jnp.float32)
    h = x_ref[...] + jnp.maximum(pre + b1_ref[0:1, :], 0.0)
    ff = jnp.maximum(jnp.dot(h, wff1_ref[...], preferred_element_type=jnp.float32), 0.0)
    o_ref[...] = h + jnp.dot(ff, wff2_ref[...], preferred_element_type=jnp.float32)


@jax.jit
def _tc_ffn(x, parts, W1, b1_t, Wff1, Wff2):
    R = 1000
    grid = (N // R,)
    row_spec = pl.BlockSpec((R, D), lambda i: (i, 0))
    p0_spec = pl.BlockSpec((1, R, D), lambda i: (0, i, 0))
    p1_spec = pl.BlockSpec((1, R, D), lambda i: (1, i, 0))
    full = lambda shape: pl.BlockSpec(shape, lambda i: (0,) * len(shape))
    return pl.pallas_call(
        _tc_body,
        grid=grid,
        in_specs=[row_spec, p0_spec, p1_spec,
                  full((D, D)), full((8, D)), full((D, 2 * D)), full((2 * D, D))],
        out_specs=row_spec,
        out_shape=jax.ShapeDtypeStruct((N, D), jnp.float32),
    )(x, parts, parts, W1, b1_t, Wff1, Wff2)


def kernel(x, edge_index, W1, b1, Wff1, Wff2):
    parts = _sc_aggregate(x, edge_index.reshape(2 * E))
    b1_t = jnp.broadcast_to(b1.reshape(1, D), (8, D))
    return _tc_ffn(x, parts, W1, b1_t, Wff1, Wff2)


# speedup vs baseline: 1.0187x; 1.0187x over previous
import functools

import jax
import jax.numpy as jnp
from jax import lax
from jax.experimental import pallas as pl
from jax.experimental.pallas import tpu as pltpu
from jax.experimental.pallas import tpu_sc as plsc

N = 10000
E = 320000
D = 128

NC = 2
NS = 16
NW = NC * NS
E_PER_TILE = E // NW
CHUNK = 80
NCHUNK = E_PER_TILE // CHUNK
NPAD = 10240
ROWS_PER_TILE = NPAD // NS
WB = 32


GROUPS = ((0, 64), (64, NCHUNK - 64))
GBUF = 64 * CHUNK


def _sc_agg_body(x_hbm, ei_hbm, out_hbm,
                 acc_sh, src_v, dst_v, rows_v, wb_v, sem_g, sem_s):
    c = lax.axis_index("c")
    s = lax.axis_index("s")
    w = s * NC + c

    def _zrow(i, _):
        for j in range(D // 16):
            wb_v[i, pl.ds(j * 16, 16)] = jnp.zeros((16,), jnp.float32)
        return 0
    lax.fori_loop(0, WB, _zrow, 0)
    row0 = s * ROWS_PER_TILE
    nz = ROWS_PER_TILE // WB
    for k in range(nz):
        pltpu.async_copy(wb_v, acc_sh.at[pl.ds(row0 + k * WB, WB)], sem_s)
    e00 = w * E_PER_TILE
    pltpu.sync_copy(ei_hbm.at[pl.ds(e00, GBUF)], src_v)
    pltpu.sync_copy(ei_hbm.at[pl.ds(E + e00, GBUF)], dst_v)
    pltpu.async_copy(x_hbm.at[src_v.at[pl.ds(0, CHUNK)]], rows_v.at[0], sem_g)
    for k in range(nz):
        pltpu.make_async_copy(wb_v, acc_sh.at[pl.ds(row0, WB)], sem_s).wait()
    plsc.subcore_barrier()

    def _gather_wait(b):
        pltpu.make_async_copy(x_hbm.at[pl.ds(0, CHUNK)], rows_v.at[b], sem_g).wait()

    def _scatter_wait(b):
        pltpu.make_async_copy(rows_v.at[b], acc_sh.at[pl.ds(0, CHUNK)], sem_s).wait()

    for base, cnt in GROUPS:
        e0 = w * E_PER_TILE + base * CHUNK
        if base:
            pltpu.sync_copy(ei_hbm.at[pl.ds(e0, cnt * CHUNK)],
                            src_v.at[pl.ds(0, cnt * CHUNK)])
            pltpu.sync_copy(ei_hbm.at[pl.ds(E + e0, cnt * CHUNK)],
                            dst_v.at[pl.ds(0, cnt * CHUNK)])
            pltpu.async_copy(x_hbm.at[src_v.at[pl.ds(0, CHUNK)]], rows_v.at[0],
                             sem_g)

        def _chunk(i, _):
            b = lax.rem(i, 2)

            @pl.when(i > 0)
            def _():
                _scatter_wait(1 - b)

            @pl.when(i < cnt - 1)
            def _():
                pltpu.async_copy(
                    x_hbm.at[src_v.at[pl.ds((i + 1) * CHUNK, CHUNK)]],
                    rows_v.at[1 - b], sem_g)
            _gather_wait(b)
            pltpu.async_copy(rows_v.at[b],
                             acc_sh.at[dst_v.at[pl.ds(i * CHUNK, CHUNK)]],
                             sem_s, add=True)
            return 0
        lax.fori_loop(0, cnt, _chunk, 0)
        _scatter_wait(lax.rem(cnt - 1, 2))
    plsc.subcore_barrier()

    WBH = 128
    nw = ROWS_PER_TILE // WBH
    for k in range(nw):
        r = row0 + k * WBH
        pltpu.async_copy(acc_sh.at[pl.ds(r, WBH)], out_hbm.at[c, pl.ds(r, WBH)],
                         sem_g)
    for k in range(nw):
        pltpu.make_async_copy(acc_sh.at[pl.ds(row0, WBH)],
                              out_hbm.at[c, pl.ds(row0, WBH)], sem_g).wait()


@jax.jit
def _sc_aggregate(x, ei4):
    mesh = plsc.VectorSubcoreMesh(core_axis_name="c", subcore_axis_name="s")
    return pl.kernel(
        _sc_agg_body,
        out_type=jax.ShapeDtypeStruct((NC, NPAD, D), jnp.float32),
        mesh=mesh,
        scratch_types=[
            pltpu.VMEM_SHARED((NPAD, D), jnp.float32),
            pltpu.VMEM((GBUF,), jnp.int32),
            pltpu.VMEM((GBUF,), jnp.int32),
            pltpu.VMEM((2, CHUNK, D), jnp.float32),
            pltpu.VMEM((WB, D), jnp.float32),
            pltpu.SemaphoreType.DMA,
            pltpu.SemaphoreType.DMA,
        ],
    )(x, ei4)


def _tc_body(x_ref, p0_ref, p1_ref, w1_ref, b1_ref, wff1_ref, wff2_ref, o_ref):
    agg = p0_ref[0] + p1_ref[0]
    pre = jnp.dot(agg, w1_ref[...], preferred_element_type=jnp.float32)
    h = x_ref[...] + jnp.maximum(pre + b1_ref[0:1, :], 0.0)
    ff = jnp.maximum(jnp.dot(h, wff1_ref[...], preferred_element_type=jnp.float32), 0.0)
    o_ref[...] = h + jnp.dot(ff, wff2_ref[...], preferred_element_type=jnp.float32)


@jax.jit
def _tc_ffn(x, parts, W1, b1_t, Wff1, Wff2):
    R = 2000
    grid = (N // R,)
    row_spec = pl.BlockSpec((R, D), lambda i: (i, 0))
    p0_spec = pl.BlockSpec((1, R, D), lambda i: (0, i, 0))
    p1_spec = pl.BlockSpec((1, R, D), lambda i: (1, i, 0))
    full = lambda shape: pl.BlockSpec(shape, lambda i: (0,) * len(shape))
    return pl.pallas_call(
        _tc_body,
        grid=grid,
        in_specs=[row_spec, p0_spec, p1_spec,
                  full((D, D)), full((8, D)), full((D, 2 * D)), full((2 * D, D))],
        out_specs=row_spec,
        out_shape=jax.ShapeDtypeStruct((N, D), jnp.float32),
    )(x, parts, parts, W1, b1_t, Wff1, Wff2)


def kernel(x, edge_index, W1, b1, Wff1, Wff2):
    parts = _sc_aggregate(x, edge_index.reshape(2 * E))
    b1_t = jnp.broadcast_to(b1.reshape(1, D), (8, D))
    return _tc_ffn(x, parts, W1, b1_t, Wff1, Wff2)


# speedup vs baseline: 1.0209x; 1.0022x over previous
import jax
import jax.numpy as jnp
from jax import lax
from jax.experimental import pallas as pl
from jax.experimental.pallas import tpu as pltpu
from jax.experimental.pallas import tpu_sc as plsc

N = 10000
E = 320000
D = 128

NC = 2
NS = 16
NW = NC * NS
E_PER_TILE = E // NW
CHUNK = 80
NCHUNK = E_PER_TILE // CHUNK
NPAD = 10240
ROWS_PER_TILE = NPAD // NS
WB = 32


GROUPS = ((0, 64), (64, NCHUNK - 64))
GBUF = 64 * CHUNK


def _sc_agg_body(x_hbm, ei_hbm, out_hbm,
                 acc_sh, src_v, dst_v, rows_v, wb_v, sem_g, sem_s):
    c = lax.axis_index("c")
    s = lax.axis_index("s")
    w = s * NC + c

    def _zrow(i, _):
        for j in range(D // 16):
            wb_v[i, pl.ds(j * 16, 16)] = jnp.zeros((16,), jnp.float32)
        return 0
    lax.fori_loop(0, WB, _zrow, 0)
    row0 = s * ROWS_PER_TILE
    nz = ROWS_PER_TILE // WB
    for k in range(nz):
        pltpu.async_copy(wb_v, acc_sh.at[pl.ds(row0 + k * WB, WB)], sem_s)
    e00 = w * E_PER_TILE
    pltpu.sync_copy(ei_hbm.at[pl.ds(e00, GBUF)], src_v)
    pltpu.sync_copy(ei_hbm.at[pl.ds(E + e00, GBUF)], dst_v)
    pltpu.async_copy(x_hbm.at[src_v.at[pl.ds(0, CHUNK)]], rows_v.at[0], sem_g)
    for k in range(nz):
        pltpu.make_async_copy(wb_v, acc_sh.at[pl.ds(row0, WB)], sem_s).wait()
    plsc.subcore_barrier()

    def _gather_wait(b):
        pltpu.make_async_copy(x_hbm.at[pl.ds(0, CHUNK)], rows_v.at[b], sem_g).wait()

    def _scatter_wait(b):
        pltpu.make_async_copy(rows_v.at[b], acc_sh.at[pl.ds(0, CHUNK)], sem_s).wait()

    for base, cnt in GROUPS:
        e0 = w * E_PER_TILE + base * CHUNK
        if base:
            pltpu.sync_copy(ei_hbm.at[pl.ds(e0, cnt * CHUNK)],
                            src_v.at[pl.ds(0, cnt * CHUNK)])
            pltpu.sync_copy(ei_hbm.at[pl.ds(E + e0, cnt * CHUNK)],
                            dst_v.at[pl.ds(0, cnt * CHUNK)])
            pltpu.async_copy(x_hbm.at[src_v.at[pl.ds(0, CHUNK)]], rows_v.at[0],
                             sem_g)

        def _chunk(i, _):
            b = lax.rem(i, 2)

            @pl.when(i > 0)
            def _():
                _scatter_wait(1 - b)

            @pl.when(i < cnt - 1)
            def _():
                pltpu.async_copy(
                    x_hbm.at[src_v.at[pl.ds((i + 1) * CHUNK, CHUNK)]],
                    rows_v.at[1 - b], sem_g)
            _gather_wait(b)
            pltpu.async_copy(rows_v.at[b],
                             acc_sh.at[dst_v.at[pl.ds(i * CHUNK, CHUNK)]],
                             sem_s, add=True)
            return 0
        lax.fori_loop(0, cnt, _chunk, 0)
        _scatter_wait(lax.rem(cnt - 1, 2))
    plsc.subcore_barrier()

    WBH = 128
    nw = ROWS_PER_TILE // WBH
    for k in range(nw):
        r = row0 + k * WBH
        pltpu.async_copy(acc_sh.at[pl.ds(r, WBH)], out_hbm.at[c, pl.ds(r, WBH)],
                         sem_g)
    for k in range(nw):
        pltpu.make_async_copy(acc_sh.at[pl.ds(row0, WBH)],
                              out_hbm.at[c, pl.ds(row0, WBH)], sem_g).wait()


@jax.jit
def _sc_aggregate(x, ei_flat):
    mesh = plsc.VectorSubcoreMesh(core_axis_name="c", subcore_axis_name="s")
    return pl.kernel(
        _sc_agg_body,
        out_type=jax.ShapeDtypeStruct((NC, NPAD, D), jnp.float32),
        mesh=mesh,
        scratch_types=[
            pltpu.VMEM_SHARED((NPAD, D), jnp.float32),
            pltpu.VMEM((GBUF,), jnp.int32),
            pltpu.VMEM((GBUF,), jnp.int32),
            pltpu.VMEM((2, CHUNK, D), jnp.float32),
            pltpu.VMEM((WB, D), jnp.float32),
            pltpu.SemaphoreType.DMA,
            pltpu.SemaphoreType.DMA,
        ],
    )(x, ei_flat)


def _tc_body(x_ref, p0_ref, p1_ref, w1_ref, b1_ref, wff1_ref, wff2_ref, o_ref):
    agg = p0_ref[0] + p1_ref[0]
    pre = jnp.dot(agg, w1_ref[...], preferred_element_type=jnp.float32)
    h = x_ref[...] + jnp.maximum(pre + b1_ref[0:1, :], 0.0)
    ff = jnp.maximum(jnp.dot(h, wff1_ref[...], preferred_element_type=jnp.float32), 0.0)
    o_ref[...] = h + jnp.dot(ff, wff2_ref[...], preferred_element_type=jnp.float32)


@jax.jit
def _tc_ffn(x, parts, W1, b1_t, Wff1, Wff2):
    R = 2000
    grid = (N // R,)
    row_spec = pl.BlockSpec((R, D), lambda i: (i, 0))
    p0_spec = pl.BlockSpec((1, R, D), lambda i: (0, i, 0))
    p1_spec = pl.BlockSpec((1, R, D), lambda i: (1, i, 0))
    full = lambda shape: pl.BlockSpec(shape, lambda i: (0,) * len(shape))
    return pl.pallas_call(
        _tc_body,
        grid=grid,
        in_specs=[row_spec, p0_spec, p1_spec,
                  full((D, D)), full((8, D)), full((D, 2 * D)), full((2 * D, D))],
        out_specs=row_spec,
        out_shape=jax.ShapeDtypeStruct((N, D), jnp.float32),
    )(x, parts, parts, W1, b1_t, Wff1, Wff2)


def kernel(x, edge_index, W1, b1, Wff1, Wff2):
    parts = _sc_aggregate(x, edge_index.reshape(2 * E))
    b1_t = jnp.broadcast_to(b1.reshape(1, D), (8, D))
    return _tc_ffn(x, parts, W1, b1_t, Wff1, Wff2)
